# baseline (speedup 1.0000x reference)
import jax
import jax.numpy as jnp
from jax import lax
from jax.experimental import pallas as pl
from jax.experimental.pallas import tpu as pltpu
from jax.experimental.pallas import tpu_sc as plsc

N = 100000
R = 8
D = 128
ROW = R * D
NC = 2
NS = 16
NW = NC * NS
L = 16
CH = 32
NCHUNK = N // CH


def _sc_body(x_hbm, out_hbm, in_v, out_v):
    wid = lax.axis_index("s") * NC + lax.axis_index("c")
    lanes = lax.broadcasted_iota(jnp.int32, (L,), 0)
    nchunks_w = (NCHUNK - wid + NW - 1) // NW

    def chunk_body(i, carry):
        c = wid + i * NW
        pltpu.sync_copy(x_hbm.at[pl.ds(c * (CH * ROW), CH * ROW)], in_v)
        for g in range(CH // L):
            rowbase = (lanes + g * L) * ROW
            obase = (lanes + g * L) * D

            def norm_body(f, st):
                col = st[0]
                accs = list(st[1:])
                for r in range(R):
                    v = plsc.load_gather(in_v, [col + (r * D)])
                    accs[r] = accs[r] + v * v
                return (col + 1,) + tuple(accs)

            zero_f = jnp.zeros((L,), jnp.float32)
            st = lax.fori_loop(0, D, norm_body,
                               (rowbase,) + (zero_f,) * R, unroll=4)
            accs = st[1:]
            best = jnp.zeros((L,), jnp.int32)
            bestv = accs[0]
            for r in range(1, R):
                m = accs[r] > bestv
                bestv = jnp.where(m, accs[r], bestv)
                best = jnp.where(m, jnp.full((L,), r, jnp.int32), best)

            def copy_body(f, st):
                col, ocol = st
                v = plsc.load_gather(in_v, [col])
                plsc.store_scatter(out_v, [ocol], v)
                return (col + 1, ocol + 1)

            lax.fori_loop(0, D, copy_body,
                          (rowbase + best * D, obase), unroll=4)
        pltpu.sync_copy(out_v, out_hbm.at[pl.ds(c * (CH * D), CH * D)])
        return carry

    lax.fori_loop(0, nchunks_w, chunk_body, 0)


def kernel(inputs):
    x = inputs.reshape(N * ROW)
    mesh = plsc.VectorSubcoreMesh(core_axis_name="c", subcore_axis_name="s")
    f = pl.kernel(
        _sc_body, mesh=mesh,
        out_type=jax.ShapeDtypeStruct((N * D,), jnp.float32),
        scratch_types=[
            pltpu.VMEM((CH * ROW,), jnp.float32),
            pltpu.VMEM((CH * D,), jnp.float32),
        ],
        compiler_params=pltpu.CompilerParams(needs_layout_passes=False),
    )
    return f(x).reshape(N, D)
